# baseline (speedup 1.0000x reference)
import functools

import jax
import jax.numpy as jnp
from jax import lax
from jax.experimental import pallas as pl
from jax.experimental.pallas import tpu as pltpu
from jax.experimental.pallas import tpu_sc as plsc

L = 16
NC = 2
NS = 16
NW = NC * NS


def _rsqrt_vec(x):
    ix = plsc.bitcast(x, jnp.int32)
    iy = jnp.int32(0x5F3759DF) - lax.shift_right_logical(ix, 1)
    y = plsc.bitcast(iy, jnp.float32)
    for _ in range(3):
        y = y * (1.5 - 0.5 * x * y * y)
    return y


def _clip_scale(ssv):
    return jnp.minimum(1.0, _rsqrt_vec(jnp.maximum(ssv, 1e-30)))


def _make_sc_kernel(B, F, K):
    IPW = B // NW
    NCH = K // L
    RPI = F * F
    mesh = plsc.VectorSubcoreMesh(core_axis_name="c", subcore_axis_name="s")

    @functools.partial(
        pl.kernel,
        out_type=jax.ShapeDtypeStruct((B,), jnp.float32),
        mesh=mesh,
        scratch_types=[
            pltpu.VMEM((IPW * RPI,), jnp.int32),
            pltpu.VMEM((IPW * F,), jnp.int32),
            pltpu.VMEM((IPW,), jnp.int32),
            pltpu.VMEM((IPW,), jnp.int32),
            pltpu.VMEM((IPW, K), jnp.float32),
            pltpu.VMEM((IPW, K), jnp.float32),
            pltpu.VMEM((2, RPI, K), jnp.float32),
            pltpu.VMEM((2, F, K), jnp.float32),
            pltpu.VMEM((IPW,), jnp.float32),
            pltpu.VMEM((IPW,), jnp.float32),
            pltpu.VMEM((IPW,), jnp.float32),
            pltpu.VMEM(((F + 2) * L,), jnp.float32),
            pltpu.SemaphoreType.DMA,
            pltpu.SemaphoreType.DMA,
            pltpu.SemaphoreType.DMA,
        ],
        compiler_params=pltpu.CompilerParams(needs_layout_passes=False),
    )
    def sc_kernel(e_hbm, u_tab_hbm, u_hbm, i_hbm, adj1_hbm, adj2_hbm,
                  out_hbm, idx2_v, idx1_v, i_idx, u_idx, irows_v, urows_v,
                  nb_v, tg_v, iscale_v, uscale_v,
                  out_v, red_v, sem_pro, sem_g0, sem_g1):
        sems = (sem_g0, sem_g1)
        wid = lax.axis_index("s") * NC + lax.axis_index("c")
        ibase = wid * IPW

        pltpu.sync_copy(adj2_hbm.at[pl.ds(ibase * RPI, IPW * RPI)], idx2_v)
        pltpu.sync_copy(adj1_hbm.at[pl.ds(ibase * F, IPW * F)], idx1_v)
        pltpu.sync_copy(i_hbm.at[pl.ds(ibase, IPW)], i_idx)
        pltpu.sync_copy(u_hbm.at[pl.ds(ibase, IPW)], u_idx)
        cp_i = pltpu.async_copy(e_hbm.at[i_idx], irows_v, sem_pro)
        cp_u = pltpu.async_copy(u_tab_hbm.at[u_idx], urows_v, sem_pro)

        def issue(li, d):
            o2 = pl.multiple_of(li * RPI, 8)
            for h in range(RPI // 128):
                pltpu.async_copy(
                    e_hbm.at[idx2_v.at[pl.ds(o2 + h * 128, 128)]],
                    nb_v.at[d, pl.ds(h * 128, 128)], sems[d])
            o1 = pl.multiple_of(li * F, 8)
            pltpu.async_copy(e_hbm.at[idx1_v.at[pl.ds(o1, F)]],
                             tg_v.at[d], sems[d])

        def drain(d):
            for h in range(RPI // 128):
                pltpu.make_async_copy(
                    e_hbm.at[idx2_v.at[pl.ds(h * 128, 128)]],
                    nb_v.at[d, pl.ds(h * 128, 128)], sems[d]).wait()
            pltpu.make_async_copy(e_hbm.at[idx1_v.at[pl.ds(0, F)]],
                                  tg_v.at[d], sems[d]).wait()

        zero = jnp.zeros((L,), jnp.float32)
        lanes = lax.iota(jnp.int32, L)

        def splat(ref, j):
            return plsc.load_gather(
                ref, [jnp.broadcast_to(j, (L,)).astype(jnp.int32)])

        def tree_at(v, slot):
            base = slot * L
            red_v[pl.ds(base, L)] = v
            v = v + plsc.load_gather(red_v, [(lanes ^ 8) + base])
            for sh in (4, 2, 1):
                red_v[pl.ds(base, L)] = v
                v = v + plsc.load_gather(red_v, [(lanes ^ sh) + base])
            return v

        def row_scale(v_chunks, slot):
            ssc = v_chunks[0] * v_chunks[0]
            for c in range(1, NCH):
                ssc = ssc + v_chunks[c] * v_chunks[c]
            return _clip_scale(tree_at(ssc, slot))

        def compute(li, d):
            def f_body(f, SQ):
                s = [zero] * NCH
                q = [zero] * NCH
                for j in range(F):
                    row = f * F + j
                    v = [nb_v[d, row, pl.ds(c * L, L)] for c in range(NCH)]
                    wj = row_scale(v, j)
                    w2j = wj * wj
                    for c in range(NCH):
                        s[c] = s[c] + v[c] * wj
                        q[c] = q[c] + (v[c] * v[c]) * w2j
                t = [tg_v[d, f, pl.ds(c * L, L)] for c in range(NCH)]
                wf = row_scale(t, F)
                out = []
                for c in range(NCH):
                    a = s[c] * s[c] - q[c] + t[c] * wf
                    out.append(SQ[c] + a)
                for c in range(NCH):
                    a = s[c] * s[c] - q[c] + t[c] * wf
                    out.append(SQ[NCH + c] + a * a)
                return tuple(out)

            SQ = lax.fori_loop(0, F, f_body, tuple([zero] * (2 * NCH)))

            wi = splat(iscale_v, li)
            acc_dot = zero
            for c in range(NCH):
                agg1 = SQ[c] * SQ[c] - SQ[NCH + c]
                itemv = agg1 + irows_v[li, pl.ds(c * L, L)] * wi
                acc_dot = acc_dot + itemv * urows_v[li, pl.ds(c * L, L)]
            dotv = tree_at(acc_dot, F + 1)
            wu = splat(uscale_v, li)
            x = dotv * wu
            sig = 1.0 / (1.0 + jnp.exp(-x))
            plsc.store_scatter(out_v, [jnp.full((L,), li, jnp.int32)],
                               sig, mask=lanes == 0)

        issue(0, 0)
        cp_i.wait()
        cp_u.wait()

        for r in range(IPW):
            iv = [irows_v[r, pl.ds(c * L, L)] for c in range(NCH)]
            plsc.store_scatter(iscale_v, [jnp.full((L,), r, jnp.int32)],
                               row_scale(iv, r % F), mask=lanes == 0)
            uv = [urows_v[r, pl.ds(c * L, L)] for c in range(NCH)]
            plsc.store_scatter(uscale_v, [jnp.full((L,), r, jnp.int32)],
                               row_scale(uv, r % F), mask=lanes == 0)

        def body2(it2, carry):
            a = it2 * 2
            issue(a + 1, 1)
            drain(0)
            compute(a, 0)

            @pl.when(a + 2 < IPW)
            def _():
                issue(a + 2, 0)

            drain(1)
            compute(a + 1, 1)
            return carry

        lax.fori_loop(0, IPW // 2, body2, 0)

        pltpu.sync_copy(out_v, out_hbm.at[pl.ds(ibase, IPW)])

    return sc_kernel


def kernel(u, i, adj1, adj2, E, U, Wa, ba, Wh, bh):
    B, F = adj1.shape
    K = E.shape[1]
    sck = _make_sc_kernel(B, F, K)
    return sck(E, U, u.astype(jnp.int32), i.astype(jnp.int32),
               adj1.reshape(-1).astype(jnp.int32),
               adj2.reshape(-1).astype(jnp.int32))


# speedup vs baseline: 1.9709x; 1.9709x over previous
import functools

import jax
import jax.numpy as jnp
from jax import lax
from jax.experimental import pallas as pl
from jax.experimental.pallas import tpu as pltpu
from jax.experimental.pallas import tpu_sc as plsc

L = 16
NC = 2
NS = 16
NW = NC * NS


def _rsqrt_vec(x):
    ix = plsc.bitcast(x, jnp.int32)
    iy = jnp.int32(0x5F3759DF) - lax.shift_right_logical(ix, 1)
    y = plsc.bitcast(iy, jnp.float32)
    for _ in range(3):
        y = y * (1.5 - 0.5 * x * y * y)
    return y


def _clip_scale(ssv):
    return jnp.minimum(1.0, _rsqrt_vec(jnp.maximum(ssv, 1e-30)))


def _make_sc_kernel(B, F, K):
    IPW = B // NW
    NCH = K // L
    RPI = F * F
    mesh = plsc.VectorSubcoreMesh(core_axis_name="c", subcore_axis_name="s")

    @functools.partial(
        pl.kernel,
        out_type=jax.ShapeDtypeStruct((B,), jnp.float32),
        mesh=mesh,
        scratch_types=[
            pltpu.VMEM((IPW * RPI,), jnp.int32),
            pltpu.VMEM((IPW * F,), jnp.int32),
            pltpu.VMEM((IPW,), jnp.int32),
            pltpu.VMEM((IPW,), jnp.int32),
            pltpu.VMEM((IPW, K), jnp.float32),
            pltpu.VMEM((IPW, K), jnp.float32),
            pltpu.VMEM((2, RPI, K), jnp.float32),
            pltpu.VMEM((2, F, K), jnp.float32),
            pltpu.VMEM((IPW,), jnp.float32),
            pltpu.VMEM((IPW,), jnp.float32),
            pltpu.VMEM((IPW,), jnp.float32),
            pltpu.VMEM(((F + 2) * L,), jnp.float32),
            pltpu.VMEM((L,), jnp.float32),
            pltpu.SemaphoreType.DMA,
            pltpu.SemaphoreType.DMA,
            pltpu.SemaphoreType.DMA,
        ],
        compiler_params=pltpu.CompilerParams(needs_layout_passes=False),
    )
    def sc_kernel(e_hbm, u_tab_hbm, u_hbm, i_hbm, adj1_hbm, adj2_hbm,
                  out_hbm, idx2_v, idx1_v, i_idx, u_idx, irows_v, urows_v,
                  nb_v, tg_v, iscale_v, uscale_v,
                  out_v, red_v, wt_v, sem_pro, sem_g0, sem_g1):
        sems = (sem_g0, sem_g1)
        wid = lax.axis_index("s") * NC + lax.axis_index("c")
        ibase = wid * IPW

        pltpu.sync_copy(adj2_hbm.at[pl.ds(ibase * RPI, IPW * RPI)], idx2_v)
        pltpu.sync_copy(adj1_hbm.at[pl.ds(ibase * F, IPW * F)], idx1_v)
        pltpu.sync_copy(i_hbm.at[pl.ds(ibase, IPW)], i_idx)
        pltpu.sync_copy(u_hbm.at[pl.ds(ibase, IPW)], u_idx)
        cp_i = pltpu.async_copy(e_hbm.at[i_idx], irows_v, sem_pro)
        cp_u = pltpu.async_copy(u_tab_hbm.at[u_idx], urows_v, sem_pro)

        def issue(li, d):
            o2 = pl.multiple_of(li * RPI, 8)
            for h in range(RPI // 128):
                pltpu.async_copy(
                    e_hbm.at[idx2_v.at[pl.ds(o2 + h * 128, 128)]],
                    nb_v.at[d, pl.ds(h * 128, 128)], sems[d])
            o1 = pl.multiple_of(li * F, 8)
            pltpu.async_copy(e_hbm.at[idx1_v.at[pl.ds(o1, F)]],
                             tg_v.at[d], sems[d])

        def drain(d):
            for h in range(RPI // 128):
                pltpu.make_async_copy(
                    e_hbm.at[idx2_v.at[pl.ds(h * 128, 128)]],
                    nb_v.at[d, pl.ds(h * 128, 128)], sems[d]).wait()
            pltpu.make_async_copy(e_hbm.at[idx1_v.at[pl.ds(0, F)]],
                                  tg_v.at[d], sems[d]).wait()

        zero = jnp.zeros((L,), jnp.float32)
        lanes = lax.iota(jnp.int32, L)

        lanes16 = lanes * L

        def splat(ref, j):
            return plsc.load_gather(
                ref, [jnp.broadcast_to(j, (L,)).astype(jnp.int32)])

        def tree_at(v, slot):
            base = slot * L
            for sh in (8, 4, 2, 1):
                red_v[pl.ds(base, L)] = v
                v = v + plsc.load_gather(red_v, [(lanes ^ sh) + base])
            return v

        def group_scales(rows):
            for j, v in enumerate(rows):
                ssc = v[0] * v[0]
                for c in range(1, NCH):
                    ssc = ssc + v[c] * v[c]
                red_v[pl.ds(j * L, L)] = ssc
            acc = plsc.load_gather(red_v, [lanes16])
            for c in range(1, L):
                acc = acc + plsc.load_gather(red_v, [lanes16 + c])
            return _clip_scale(acc)

        def compute(li, d):
            t_rows = [[tg_v[d, j, pl.ds(c * L, L)] for c in range(NCH)]
                      for j in range(F)]
            wt_v[...] = group_scales(t_rows)

            def f_body(f, SQ):
                rows = [[nb_v[d, f * F + j, pl.ds(c * L, L)]
                         for c in range(NCH)] for j in range(F)]
                w = group_scales(rows)
                red_v[pl.ds(F * L, L)] = w
                red_v[pl.ds((F + 1) * L, L)] = w * w
                s = [zero] * NCH
                q = [zero] * NCH
                for j in range(F):
                    row = f * F + j
                    wj = splat(red_v, F * L + j)
                    w2j = splat(red_v, (F + 1) * L + j)
                    for c in range(NCH):
                        v = nb_v[d, row, pl.ds(c * L, L)]
                        s[c] = s[c] + v * wj
                        q[c] = q[c] + (v * v) * w2j
                wf = splat(wt_v, f)
                avals = []
                for c in range(NCH):
                    t = tg_v[d, f, pl.ds(c * L, L)]
                    avals.append(s[c] * s[c] - q[c] + t * wf)
                res = []
                for c in range(NCH):
                    res.append(SQ[c] + avals[c])
                for c in range(NCH):
                    res.append(SQ[NCH + c] + avals[c] * avals[c])
                return tuple(res)

            SQ = lax.fori_loop(0, F, f_body, tuple([zero] * (2 * NCH)))

            wi = splat(iscale_v, li)
            acc_dot = zero
            for c in range(NCH):
                agg1 = SQ[c] * SQ[c] - SQ[NCH + c]
                itemv = agg1 + irows_v[li, pl.ds(c * L, L)] * wi
                acc_dot = acc_dot + itemv * urows_v[li, pl.ds(c * L, L)]
            dotv = tree_at(acc_dot, 0)
            wu = splat(uscale_v, li)
            x = dotv * wu
            sig = 1.0 / (1.0 + jnp.exp(-x))
            plsc.store_scatter(out_v, [jnp.full((L,), li, jnp.int32)],
                               sig, mask=lanes == 0)

        issue(0, 0)
        cp_i.wait()
        cp_u.wait()

        for g in range(IPW // L):
            iv = [[irows_v[g * L + j, pl.ds(c * L, L)] for c in range(NCH)]
                  for j in range(L)]
            iscale_v[pl.ds(g * L, L)] = group_scales(iv)
            uv = [[urows_v[g * L + j, pl.ds(c * L, L)] for c in range(NCH)]
                  for j in range(L)]
            uscale_v[pl.ds(g * L, L)] = group_scales(uv)

        def body2(it2, carry):
            a = it2 * 2
            issue(a + 1, 1)
            drain(0)
            compute(a, 0)

            @pl.when(a + 2 < IPW)
            def _():
                issue(a + 2, 0)

            drain(1)
            compute(a + 1, 1)
            return carry

        lax.fori_loop(0, IPW // 2, body2, 0)

        pltpu.sync_copy(out_v, out_hbm.at[pl.ds(ibase, IPW)])

    return sc_kernel


def kernel(u, i, adj1, adj2, E, U, Wa, ba, Wh, bh):
    B, F = adj1.shape
    K = E.shape[1]
    sck = _make_sc_kernel(B, F, K)
    return sck(E, U, u.astype(jnp.int32), i.astype(jnp.int32),
               adj1.reshape(-1).astype(jnp.int32),
               adj2.reshape(-1).astype(jnp.int32))


# speedup vs baseline: 2.9043x; 1.4736x over previous
import functools

import jax
import jax.numpy as jnp
from jax import lax
from jax.experimental import pallas as pl
from jax.experimental.pallas import tpu as pltpu
from jax.experimental.pallas import tpu_sc as plsc

L = 16
NC = 2
NS = 16
NW = NC * NS


def _normalize_body(x_ref, o_ref):
    x = x_ref[...]
    ss = jnp.sum(x * x, axis=1, keepdims=True)
    scale = jnp.minimum(1.0, 1.0 / jnp.maximum(jnp.sqrt(ss), 1e-7))
    o_ref[...] = x * scale


def _normalize_table(E):
    n, k = E.shape
    bs = 4000 if n % 4000 == 0 else (1000 if n % 1000 == 0 else 8)
    assert n % bs == 0 and bs % 8 == 0
    return pl.pallas_call(
        _normalize_body,
        grid=(n // bs,),
        in_specs=[pl.BlockSpec((bs, k), lambda g: (g, 0))],
        out_specs=pl.BlockSpec((bs, k), lambda g: (g, 0)),
        out_shape=jax.ShapeDtypeStruct((n, k), jnp.float32),
    )(E)


def _rsqrt_vec(x):
    ix = plsc.bitcast(x, jnp.int32)
    iy = jnp.int32(0x5F3759DF) - lax.shift_right_logical(ix, 1)
    y = plsc.bitcast(iy, jnp.float32)
    for _ in range(3):
        y = y * (1.5 - 0.5 * x * y * y)
    return y


def _make_sc_kernel(B, F, K):
    IPW = B // NW
    NCH = K // L
    RPI = F * F
    mesh = plsc.VectorSubcoreMesh(core_axis_name="c", subcore_axis_name="s")

    @functools.partial(
        pl.kernel,
        out_type=jax.ShapeDtypeStruct((B,), jnp.float32),
        mesh=mesh,
        scratch_types=[
            pltpu.VMEM((IPW * RPI,), jnp.int32),
            pltpu.VMEM((IPW * F,), jnp.int32),
            pltpu.VMEM((IPW,), jnp.int32),
            pltpu.VMEM((IPW,), jnp.int32),
            pltpu.VMEM((IPW, K), jnp.float32),
            pltpu.VMEM((IPW, K), jnp.float32),
            pltpu.VMEM((2, RPI, K), jnp.float32),
            pltpu.VMEM((2, F, K), jnp.float32),
            pltpu.VMEM((IPW,), jnp.float32),
            pltpu.VMEM((L,), jnp.float32),
            pltpu.SemaphoreType.DMA,
            pltpu.SemaphoreType.DMA,
            pltpu.SemaphoreType.DMA,
        ],
        compiler_params=pltpu.CompilerParams(needs_layout_passes=False),
    )
    def sc_kernel(en_hbm, u_tab_hbm, u_hbm, i_hbm, adj1_hbm, adj2_hbm,
                  out_hbm, idx2_v, idx1_v, i_idx, u_idx, irows_v, urows_v,
                  nb_v, tg_v, out_v, red_v, sem_pro, sem_g0, sem_g1):
        sems = (sem_g0, sem_g1)
        wid = lax.axis_index("s") * NC + lax.axis_index("c")
        ibase = wid * IPW

        pltpu.sync_copy(adj2_hbm.at[pl.ds(ibase * RPI, IPW * RPI)], idx2_v)
        pltpu.sync_copy(adj1_hbm.at[pl.ds(ibase * F, IPW * F)], idx1_v)
        pltpu.sync_copy(i_hbm.at[pl.ds(ibase, IPW)], i_idx)
        pltpu.sync_copy(u_hbm.at[pl.ds(ibase, IPW)], u_idx)
        cp_i = pltpu.async_copy(en_hbm.at[i_idx], irows_v, sem_pro)
        cp_u = pltpu.async_copy(u_tab_hbm.at[u_idx], urows_v, sem_pro)

        def issue(li, d):
            o2 = pl.multiple_of(li * RPI, 8)
            for h in range(RPI // 128):
                pltpu.async_copy(
                    en_hbm.at[idx2_v.at[pl.ds(o2 + h * 128, 128)]],
                    nb_v.at[d, pl.ds(h * 128, 128)], sems[d])
            o1 = pl.multiple_of(li * F, 8)
            pltpu.async_copy(en_hbm.at[idx1_v.at[pl.ds(o1, F)]],
                             tg_v.at[d], sems[d])

        def drain(d):
            for h in range(RPI // 128):
                pltpu.make_async_copy(
                    en_hbm.at[idx2_v.at[pl.ds(h * 128, 128)]],
                    nb_v.at[d, pl.ds(h * 128, 128)], sems[d]).wait()
            pltpu.make_async_copy(en_hbm.at[idx1_v.at[pl.ds(0, F)]],
                                  tg_v.at[d], sems[d]).wait()

        zero = jnp.zeros((L,), jnp.float32)
        lanes = lax.iota(jnp.int32, L)

        def reduce_tree(v):
            for sh in (8, 4, 2, 1):
                red_v[...] = v
                v = v + plsc.load_gather(red_v, [lanes ^ sh])
            return v

        def compute(li, d):
            def f_body(f, SQ):
                s = [zero] * NCH
                q = [zero] * NCH
                for j in range(F):
                    row = f * F + j
                    for c in range(NCH):
                        v = nb_v[d, row, pl.ds(c * L, L)]
                        s[c] = s[c] + v
                        q[c] = q[c] + v * v
                avals = []
                for c in range(NCH):
                    t = tg_v[d, f, pl.ds(c * L, L)]
                    avals.append(s[c] * s[c] - q[c] + t)
                res = []
                for c in range(NCH):
                    res.append(SQ[c] + avals[c])
                for c in range(NCH):
                    res.append(SQ[NCH + c] + avals[c] * avals[c])
                return tuple(res)

            SQ = lax.fori_loop(0, F, f_body, tuple([zero] * (2 * NCH)))

            acc_dot = zero
            acc_ss = zero
            for c in range(NCH):
                agg1 = SQ[c] * SQ[c] - SQ[NCH + c]
                itemv = agg1 + irows_v[li, pl.ds(c * L, L)]
                uv = urows_v[li, pl.ds(c * L, L)]
                acc_dot = acc_dot + itemv * uv
                acc_ss = acc_ss + uv * uv
            dotv = reduce_tree(acc_dot)
            ssv = jnp.maximum(reduce_tree(acc_ss), 1e-30)
            scale = jnp.minimum(1.0, _rsqrt_vec(ssv))
            x = dotv * scale
            sig = 1.0 / (1.0 + jnp.exp(-x))
            plsc.store_scatter(out_v, [jnp.full((L,), li, jnp.int32)],
                               sig, mask=lanes == 0)

        issue(0, 0)
        cp_i.wait()
        cp_u.wait()

        def body2(it2, carry):
            a = it2 * 2
            issue(a + 1, 1)
            drain(0)
            compute(a, 0)

            @pl.when(a + 2 < IPW)
            def _():
                issue(a + 2, 0)

            drain(1)
            compute(a + 1, 1)
            return carry

        lax.fori_loop(0, IPW // 2, body2, 0)

        pltpu.sync_copy(out_v, out_hbm.at[pl.ds(ibase, IPW)])

    return sc_kernel


def kernel(u, i, adj1, adj2, E, U, Wa, ba, Wh, bh):
    B, F = adj1.shape
    K = E.shape[1]
    En = _normalize_table(E)
    sck = _make_sc_kernel(B, F, K)
    return sck(En, U, u.astype(jnp.int32), i.astype(jnp.int32),
               adj1.reshape(-1).astype(jnp.int32),
               adj2.reshape(-1).astype(jnp.int32))


# speedup vs baseline: 3.0664x; 1.0558x over previous
import functools

import jax
import jax.numpy as jnp
from jax import lax
from jax.experimental import pallas as pl
from jax.experimental.pallas import tpu as pltpu
from jax.experimental.pallas import tpu_sc as plsc

L = 16
NC = 2
NS = 16
NW = NC * NS


def _normalize_body(x_ref, o_ref):
    x = x_ref[...]
    ss = jnp.sum(x * x, axis=1, keepdims=True)
    scale = jnp.minimum(1.0, 1.0 / jnp.maximum(jnp.sqrt(ss), 1e-7))
    o_ref[...] = x * scale


def _normalize_table(E):
    n, k = E.shape
    bs = 10000 if n % 10000 == 0 else (1000 if n % 1000 == 0 else 8)
    assert n % bs == 0 and bs % 8 == 0
    return pl.pallas_call(
        _normalize_body,
        grid=(n // bs,),
        in_specs=[pl.BlockSpec((bs, k), lambda g: (g, 0))],
        out_specs=pl.BlockSpec((bs, k), lambda g: (g, 0)),
        out_shape=jax.ShapeDtypeStruct((n, k), jnp.float32),
    )(E)


def _rsqrt_vec(x):
    ix = plsc.bitcast(x, jnp.int32)
    iy = jnp.int32(0x5F3759DF) - lax.shift_right_logical(ix, 1)
    y = plsc.bitcast(iy, jnp.float32)
    for _ in range(3):
        y = y * (1.5 - 0.5 * x * y * y)
    return y


def _make_sc_kernel(B, F, K):
    IPW = B // NW
    NCH = K // L
    RPI = F * F
    mesh = plsc.VectorSubcoreMesh(core_axis_name="c", subcore_axis_name="s")

    @functools.partial(
        pl.kernel,
        out_type=jax.ShapeDtypeStruct((B,), jnp.float32),
        mesh=mesh,
        scratch_types=[
            pltpu.VMEM((IPW * RPI,), jnp.int32),
            pltpu.VMEM((IPW * F,), jnp.int32),
            pltpu.VMEM((IPW,), jnp.int32),
            pltpu.VMEM((IPW,), jnp.int32),
            pltpu.VMEM((IPW, K), jnp.float32),
            pltpu.VMEM((IPW, K), jnp.float32),
            pltpu.VMEM((2, RPI, K), jnp.float32),
            pltpu.VMEM((2, F, K), jnp.float32),
            pltpu.VMEM((IPW,), jnp.float32),
            pltpu.VMEM((L,), jnp.float32),
            pltpu.SemaphoreType.DMA,
            pltpu.SemaphoreType.DMA,
            pltpu.SemaphoreType.DMA,
        ],
        compiler_params=pltpu.CompilerParams(needs_layout_passes=False),
    )
    def sc_kernel(en_hbm, u_tab_hbm, u_hbm, i_hbm, adj1_hbm, adj2_hbm,
                  out_hbm, idx2_v, idx1_v, i_idx, u_idx, irows_v, urows_v,
                  nb_v, tg_v, out_v, red_v, sem_pro, sem_g0, sem_g1):
        sems = (sem_g0, sem_g1)
        wid = lax.axis_index("s") * NC + lax.axis_index("c")
        ibase = wid * IPW

        pltpu.sync_copy(adj2_hbm.at[pl.ds(ibase * RPI, IPW * RPI)], idx2_v)
        pltpu.sync_copy(adj1_hbm.at[pl.ds(ibase * F, IPW * F)], idx1_v)
        pltpu.sync_copy(i_hbm.at[pl.ds(ibase, IPW)], i_idx)
        pltpu.sync_copy(u_hbm.at[pl.ds(ibase, IPW)], u_idx)
        cp_i = pltpu.async_copy(en_hbm.at[i_idx], irows_v, sem_pro)
        cp_u = pltpu.async_copy(u_tab_hbm.at[u_idx], urows_v, sem_pro)

        def issue(li, d):
            o2 = pl.multiple_of(li * RPI, 8)
            for h in range(RPI // 128):
                pltpu.async_copy(
                    en_hbm.at[idx2_v.at[pl.ds(o2 + h * 128, 128)]],
                    nb_v.at[d, pl.ds(h * 128, 128)], sems[d])
            o1 = pl.multiple_of(li * F, 8)
            pltpu.async_copy(en_hbm.at[idx1_v.at[pl.ds(o1, F)]],
                             tg_v.at[d], sems[d])

        def drain(d):
            for h in range(RPI // 128):
                pltpu.make_async_copy(
                    en_hbm.at[idx2_v.at[pl.ds(h * 128, 128)]],
                    nb_v.at[d, pl.ds(h * 128, 128)], sems[d]).wait()
            pltpu.make_async_copy(en_hbm.at[idx1_v.at[pl.ds(0, F)]],
                                  tg_v.at[d], sems[d]).wait()

        zero = jnp.zeros((L,), jnp.float32)
        lanes = lax.iota(jnp.int32, L)

        def reduce_tree(v):
            for sh in (8, 4, 2, 1):
                red_v[...] = v
                v = v + plsc.load_gather(red_v, [lanes ^ sh])
            return v

        def compute(li, d):
            def f_body(f, SQ):
                s = [zero] * NCH
                q = [zero] * NCH
                for j in range(F):
                    row = f * F + j
                    for c in range(NCH):
                        v = nb_v[d, row, pl.ds(c * L, L)]
                        s[c] = s[c] + v
                        q[c] = q[c] + v * v
                avals = []
                for c in range(NCH):
                    t = tg_v[d, f, pl.ds(c * L, L)]
                    avals.append(s[c] * s[c] - q[c] + t)
                res = []
                for c in range(NCH):
                    res.append(SQ[c] + avals[c])
                for c in range(NCH):
                    res.append(SQ[NCH + c] + avals[c] * avals[c])
                return tuple(res)

            SQ = lax.fori_loop(0, F, f_body, tuple([zero] * (2 * NCH)))

            acc_dot = zero
            acc_ss = zero
            for c in range(NCH):
                agg1 = SQ[c] * SQ[c] - SQ[NCH + c]
                itemv = agg1 + irows_v[li, pl.ds(c * L, L)]
                uv = urows_v[li, pl.ds(c * L, L)]
                acc_dot = acc_dot + itemv * uv
                acc_ss = acc_ss + uv * uv
            dotv = reduce_tree(acc_dot)
            ssv = jnp.maximum(reduce_tree(acc_ss), 1e-30)
            scale = jnp.minimum(1.0, _rsqrt_vec(ssv))
            x = dotv * scale
            sig = 1.0 / (1.0 + jnp.exp(-x))
            plsc.store_scatter(out_v, [jnp.full((L,), li, jnp.int32)],
                               sig, mask=lanes == 0)

        issue(0, 0)
        cp_i.wait()
        cp_u.wait()

        def body2(it2, carry):
            a = it2 * 2
            issue(a + 1, 1)
            drain(0)
            compute(a, 0)

            @pl.when(a + 2 < IPW)
            def _():
                issue(a + 2, 0)

            drain(1)
            compute(a + 1, 1)
            return carry

        lax.fori_loop(0, IPW // 2, body2, 0)

        pltpu.sync_copy(out_v, out_hbm.at[pl.ds(ibase, IPW)])

    return sc_kernel


def kernel(u, i, adj1, adj2, E, U, Wa, ba, Wh, bh):
    B, F = adj1.shape
    K = E.shape[1]
    En = _normalize_table(E)
    sck = _make_sc_kernel(B, F, K)
    return sck(En, U, u.astype(jnp.int32), i.astype(jnp.int32),
               adj1.reshape(-1).astype(jnp.int32),
               adj2.reshape(-1).astype(jnp.int32))
